# Initial kernel scaffold; baseline (speedup 1.0000x reference)
import functools

import jax
import jax.numpy as jnp
from jax import lax
from jax.experimental import pallas as pl
from jax.experimental.pallas import tpu as pltpu
from jax.experimental.pallas import tpu_sc as plsc

N = 100000
E = 3200000
DIN = 128
DOUT = 16

NC = 2
NS = 16
NW = NC * NS

ROW_BLK = 2048
N_PAD = 100352
TC_GRID = N_PAD // ROW_BLK
SEG = N_PAD // NS
ZCH = SEG // 8

EPT = 102400
E_PAD = EPT * NW
E_ROWS = E_PAD // 128
RPT = EPT // 128
CHUNK = 16
NCHUNK = RPT // CHUNK

_mesh = plsc.VectorSubcoreMesh(
    core_axis_name="c", subcore_axis_name="s", num_cores=NC, num_subcores=NS
)


def _fill(ref, nvec, value):
  v = jnp.full((16,), value, jnp.float32)
  if len(ref.shape) == 1:
    def body(i, _):
      ref[pl.ds(i * 16, 16)] = v
      return 0
  else:
    def body(i, _):
      ref[i, :] = v
      return 0
  lax.fori_loop(0, nvec, body, 0)


@functools.partial(
    pl.kernel,
    out_type=jax.ShapeDtypeStruct((NC, N_PAD), jnp.float32),
    mesh=_mesh,
    scratch_types=[
        pltpu.VMEM((CHUNK, 128), jnp.int32),
        pltpu.VMEM((128,), jnp.float32),
        pltpu.VMEM((SEG,), jnp.float32),
        pltpu.VMEM_SHARED((N_PAD,), jnp.float32),
    ],
)
def _sc_degree(dst_hbm, deg_hbm, idx_v, ones_v, stage_v, deg_sp):
  cid = lax.axis_index("c")
  sid = lax.axis_index("s")
  wid = cid * NS + sid

  _fill(ones_v, 8, 1.0)
  _fill(stage_v, SEG // 16, 0.0)
  pltpu.sync_copy(stage_v, deg_sp.at[pl.ds(sid * SEG, SEG)])
  plsc.subcore_barrier()

  tbase = wid * RPT

  def chunk_body(ci, _):
    pltpu.sync_copy(dst_hbm.at[pl.ds(tbase + ci * CHUNK, CHUNK)], idx_v)
    for j in range(CHUNK):
      pltpu.sync_copy(ones_v, deg_sp.at[idx_v.at[j]], add=True)
    return 0

  lax.fori_loop(0, NCHUNK, chunk_body, 0)
  plsc.subcore_barrier()

  pltpu.sync_copy(deg_sp.at[pl.ds(sid * SEG, SEG)], stage_v)
  pltpu.sync_copy(stage_v, deg_hbm.at[cid, pl.ds(sid * SEG, SEG)])


@functools.partial(
    pl.kernel,
    out_type=jax.ShapeDtypeStruct((NC, N_PAD, DOUT), jnp.float32),
    mesh=_mesh,
    scratch_types=[
        pltpu.VMEM((CHUNK, 128), jnp.int32),
        pltpu.VMEM((CHUNK, 128), jnp.int32),
        pltpu.VMEM((CHUNK, 128, DOUT), jnp.float32),
        pltpu.VMEM((ZCH, DOUT), jnp.float32),
        pltpu.VMEM_SHARED((N_PAD, DOUT), jnp.float32),
        pltpu.SemaphoreType.DMA,
    ],
)
def _sc_agg(hs_hbm, src_hbm, dst_hbm, acc_hbm,
            src_v, dst_v, rows_v, stage_v, acc_sp, sem):
  cid = lax.axis_index("c")
  sid = lax.axis_index("s")
  wid = cid * NS + sid

  _fill(stage_v, ZCH, 0.0)
  for k in range(8):
    pltpu.sync_copy(stage_v, acc_sp.at[pl.ds(sid * SEG + k * ZCH, ZCH)])
  plsc.subcore_barrier()

  tbase = wid * RPT

  def chunk_body(ci, _):
    r0 = tbase + ci * CHUNK
    pltpu.sync_copy(src_hbm.at[pl.ds(r0, CHUNK)], src_v)
    pltpu.sync_copy(dst_hbm.at[pl.ds(r0, CHUNK)], dst_v)
    descs = [
        pltpu.async_copy(hs_hbm.at[src_v.at[j]], rows_v.at[j], sem)
        for j in range(CHUNK)
    ]
    for d in descs:
      d.wait()
    for j in range(CHUNK):
      pltpu.sync_copy(rows_v.at[j], acc_sp.at[dst_v.at[j]], add=True)
    return 0

  lax.fori_loop(0, NCHUNK, chunk_body, 0)
  plsc.subcore_barrier()

  for k in range(8):
    pltpu.sync_copy(acc_sp.at[pl.ds(sid * SEG + k * ZCH, ZCH)], stage_v)
    pltpu.sync_copy(stage_v, acc_hbm.at[cid, pl.ds(sid * SEG + k * ZCH, ZCH)])


def _tc_first_body(x_ref, w_ref, d0_ref, d1_ref, hs_ref, dis_ref):
  z = jnp.dot(x_ref[...], w_ref[...], preferred_element_type=jnp.float32)
  d = d0_ref[...] + d1_ref[...] + 1.0
  dis = jnp.where(d > 0, lax.rsqrt(jnp.maximum(d, 1e-12)), 0.0)
  hs_ref[...] = z * dis
  dis_ref[...] = dis


def _tc_mid_body(a0_ref, a1_ref, hs_ref, dis_ref, w_ref, b_ref, out_ref):
  dis = dis_ref[...]
  h = dis * (a0_ref[...] + a1_ref[...] + hs_ref[...]) + b_ref[...]
  h = jnp.maximum(h, 0.0)
  out_ref[...] = jnp.dot(h, w_ref[...], preferred_element_type=jnp.float32) * dis


def _tc_last_body(a0_ref, a1_ref, hs_ref, dis_ref, b_ref, out_ref):
  out_ref[...] = (
      dis_ref[...] * (a0_ref[...] + a1_ref[...] + hs_ref[...]) + b_ref[...]
  )


def _rows_spec(width):
  return pl.BlockSpec((ROW_BLK, width), lambda i: (i, 0))


def _full_spec(shape):
  return pl.BlockSpec(shape, lambda i: tuple(0 for _ in shape))


_tc_params = pltpu.CompilerParams(dimension_semantics=("arbitrary",))

_tc_first = pl.pallas_call(
    _tc_first_body,
    grid=(TC_GRID,),
    in_specs=[
        _rows_spec(DIN),
        _full_spec((DIN, DOUT)),
        _rows_spec(1),
        _rows_spec(1),
    ],
    out_specs=[_rows_spec(DOUT), _rows_spec(1)],
    out_shape=[
        jax.ShapeDtypeStruct((N_PAD, DOUT), jnp.float32),
        jax.ShapeDtypeStruct((N_PAD, 1), jnp.float32),
    ],
    compiler_params=_tc_params,
)

_tc_mid = pl.pallas_call(
    _tc_mid_body,
    grid=(TC_GRID,),
    in_specs=[
        _rows_spec(DOUT),
        _rows_spec(DOUT),
        _rows_spec(DOUT),
        _rows_spec(1),
        _full_spec((DOUT, DOUT)),
        _full_spec((1, DOUT)),
    ],
    out_specs=_rows_spec(DOUT),
    out_shape=jax.ShapeDtypeStruct((N_PAD, DOUT), jnp.float32),
    compiler_params=_tc_params,
)

_tc_last = pl.pallas_call(
    _tc_last_body,
    grid=(TC_GRID,),
    in_specs=[
        _rows_spec(DOUT),
        _rows_spec(DOUT),
        _rows_spec(DOUT),
        _rows_spec(1),
        _full_spec((1, DOUT)),
    ],
    out_specs=_rows_spec(DOUT),
    out_shape=jax.ShapeDtypeStruct((N_PAD, DOUT), jnp.float32),
    compiler_params=_tc_params,
)


@jax.jit
def kernel(x, edge_index, W1, b1, W2, b2):
  pad_e = E_PAD - E
  src = jnp.concatenate(
      [edge_index[0], jnp.full((pad_e,), N, jnp.int32)]).reshape(E_ROWS, 128)
  dst = jnp.concatenate(
      [edge_index[1], jnp.full((pad_e,), N, jnp.int32)]).reshape(E_ROWS, 128)
  x_pad = jnp.pad(x, ((0, N_PAD - N), (0, 0)))

  deg = _sc_degree(dst)
  d0 = deg[0].reshape(N_PAD, 1)
  d1 = deg[1].reshape(N_PAD, 1)

  hs1, dis = _tc_first(x_pad, W1, d0, d1)

  acc1 = _sc_agg(hs1, src, dst)
  hs2 = _tc_mid(acc1[0], acc1[1], hs1, dis, W2, b1.reshape(1, DOUT))

  acc2 = _sc_agg(hs2, src, dst)
  out = _tc_last(acc2[0], acc2[1], hs2, dis, b2.reshape(1, DOUT))
  return out[:N]


# speedup vs baseline: 15.9890x; 15.9890x over previous
import functools

import jax
import jax.numpy as jnp
from jax import lax
from jax.experimental import pallas as pl
from jax.experimental.pallas import tpu as pltpu
from jax.experimental.pallas import tpu_sc as plsc

N = 100000
E = 3200000
DIN = 128
DOUT = 16

NC = 2
NS = 16

ROW_BLK = 2048
N_PAD = 100352
TC_GRID = N_PAD // ROW_BLK
H = N_PAD // 2
LTRASH = H
H_ALL = H + 16
HSEG = H // NS
DSEG = N_PAD // NS
ZCH = 784

E_PAD = 2048 * 32 * 50
E_ROWS = E_PAD // 128
RPT = E_ROWS // NS
CHUNK = 16
NCHUNK = RPT // CHUNK
DRPT = E_ROWS // (NC * NS)
DNCHUNK = DRPT // CHUNK

_mesh = plsc.VectorSubcoreMesh(
    core_axis_name="c", subcore_axis_name="s", num_cores=NC, num_subcores=NS
)
_sc_params = pltpu.CompilerParams(use_tc_tiling_on_sc=False)


def _fill(ref, nvec, value):
  v = jnp.full((16,), value, jnp.float32)
  if len(ref.shape) == 1:
    def body(i, _):
      ref[pl.ds(i * 16, 16)] = v
      return 0
  else:
    def body(i, _):
      ref[i, :] = v
      return 0
  lax.fori_loop(0, nvec, body, 0)


@functools.partial(
    pl.kernel,
    out_type=jax.ShapeDtypeStruct((NC, N_PAD), jnp.float32),
    mesh=_mesh,
    scratch_types=[
        pltpu.VMEM((CHUNK, 128), jnp.int32),
        pltpu.VMEM((128,), jnp.float32),
        pltpu.VMEM((DSEG,), jnp.float32),
        pltpu.VMEM_SHARED((N_PAD,), jnp.float32),
    ],
    compiler_params=_sc_params,
)
def _sc_degree(dst_hbm, deg_hbm, idx_v, ones_v, stage_v, deg_sp):
  cid = lax.axis_index("c")
  sid = lax.axis_index("s")
  wid = cid * NS + sid

  _fill(ones_v, 8, 1.0)
  _fill(stage_v, DSEG // 16, 0.0)
  pltpu.sync_copy(stage_v, deg_sp.at[pl.ds(sid * DSEG, DSEG)])
  plsc.subcore_barrier()

  tbase = wid * DRPT

  def chunk_body(ci, _):
    pltpu.sync_copy(dst_hbm.at[pl.ds(tbase + ci * CHUNK, CHUNK)], idx_v)
    for j in range(CHUNK):
      pltpu.sync_copy(ones_v, deg_sp.at[idx_v.at[j]], add=True)
    return 0

  lax.fori_loop(0, DNCHUNK, chunk_body, 0)
  plsc.subcore_barrier()

  pltpu.sync_copy(deg_sp.at[pl.ds(sid * DSEG, DSEG)], stage_v)
  pltpu.sync_copy(stage_v, deg_hbm.at[cid, pl.ds(sid * DSEG, DSEG)])


@functools.partial(
    pl.kernel,
    out_type=jax.ShapeDtypeStruct((NC, H, DOUT), jnp.float32),
    mesh=_mesh,
    scratch_types=[
        pltpu.VMEM((CHUNK, 128), jnp.int32),
        pltpu.VMEM((CHUNK, 128), jnp.int32),
        pltpu.VMEM((CHUNK, 128, DOUT), jnp.float32),
        pltpu.VMEM((ZCH, DOUT), jnp.float32),
        pltpu.VMEM_SHARED((H_ALL, DOUT), jnp.float32),
        pltpu.SemaphoreType.DMA,
    ],
    compiler_params=_sc_params,
)
def _sc_agg(hs_hbm, src_hbm, dst2_hbm, acc_hbm,
            src_v, dst_v, rows_v, stage_v, acc_sp, sem):
  cid = lax.axis_index("c")
  sid = lax.axis_index("s")

  _fill(stage_v, ZCH, 0.0)
  for k in range(4):
    pltpu.sync_copy(stage_v, acc_sp.at[pl.ds(sid * HSEG + k * ZCH, ZCH)])

  @pl.when(sid == 0)
  def _():
    pltpu.sync_copy(stage_v.at[pl.ds(0, 16)], acc_sp.at[pl.ds(H, 16)])

  plsc.subcore_barrier()

  tbase = sid * RPT

  def chunk_body(ci, _):
    r0 = tbase + ci * CHUNK
    pltpu.sync_copy(src_hbm.at[pl.ds(r0, CHUNK)], src_v)
    pltpu.sync_copy(dst2_hbm.at[cid, pl.ds(r0, CHUNK)], dst_v)
    descs = [
        pltpu.async_copy(hs_hbm.at[src_v.at[j]], rows_v.at[j], sem)
        for j in range(CHUNK)
    ]
    for d in descs:
      d.wait()
    for j in range(CHUNK):
      pltpu.sync_copy(rows_v.at[j], acc_sp.at[dst_v.at[j]], add=True)
    return 0

  lax.fori_loop(0, NCHUNK, chunk_body, 0)
  plsc.subcore_barrier()

  for k in range(4):
    pltpu.sync_copy(acc_sp.at[pl.ds(sid * HSEG + k * ZCH, ZCH)], stage_v)
    pltpu.sync_copy(stage_v, acc_hbm.at[cid, pl.ds(sid * HSEG + k * ZCH, ZCH)])


def _tc_remap_body(dst_ref, out_ref):
  d = dst_ref[...]
  out_ref[0, :, :] = jnp.where(d < H, d, LTRASH)
  out_ref[1, :, :] = jnp.where(d >= H, d - H, LTRASH)


def _tc_first_body(x_ref, w_ref, d0_ref, d1_ref, hs_ref, dis_ref):
  z = jnp.dot(x_ref[...], w_ref[...], preferred_element_type=jnp.float32)
  d = d0_ref[...] + d1_ref[...] + 1.0
  dis = jnp.where(d > 0, lax.rsqrt(jnp.maximum(d, 1e-12)), 0.0)
  hs_ref[...] = z * dis
  dis_ref[...] = dis


def _tc_mid_body(a_ref, hs_ref, dis_ref, w_ref, b_ref, out_ref):
  dis = dis_ref[...]
  h = dis * (a_ref[...] + hs_ref[...]) + b_ref[...]
  h = jnp.maximum(h, 0.0)
  out_ref[...] = jnp.dot(h, w_ref[...], preferred_element_type=jnp.float32) * dis


def _tc_last_body(a_ref, hs_ref, dis_ref, b_ref, out_ref):
  out_ref[...] = dis_ref[...] * (a_ref[...] + hs_ref[...]) + b_ref[...]


def _rows_spec(width):
  return pl.BlockSpec((ROW_BLK, width), lambda i: (i, 0))


def _full_spec(shape):
  return pl.BlockSpec(shape, lambda i: tuple(0 for _ in shape))


_tc_params = pltpu.CompilerParams(dimension_semantics=("arbitrary",))

EBLK = 512
_tc_remap = pl.pallas_call(
    _tc_remap_body,
    grid=(E_ROWS // EBLK,),
    in_specs=[pl.BlockSpec((EBLK, 128), lambda i: (i, 0))],
    out_specs=pl.BlockSpec((2, EBLK, 128), lambda i: (0, i, 0)),
    out_shape=jax.ShapeDtypeStruct((2, E_ROWS, 128), jnp.int32),
    compiler_params=_tc_params,
)

_tc_first = pl.pallas_call(
    _tc_first_body,
    grid=(TC_GRID,),
    in_specs=[
        _rows_spec(DIN),
        _full_spec((DIN, DOUT)),
        _rows_spec(1),
        _rows_spec(1),
    ],
    out_specs=[_rows_spec(DOUT), _rows_spec(1)],
    out_shape=[
        jax.ShapeDtypeStruct((N_PAD, DOUT), jnp.float32),
        jax.ShapeDtypeStruct((N_PAD, 1), jnp.float32),
    ],
    compiler_params=_tc_params,
)

_tc_mid = pl.pallas_call(
    _tc_mid_body,
    grid=(TC_GRID,),
    in_specs=[
        _rows_spec(DOUT),
        _rows_spec(DOUT),
        _rows_spec(1),
        _full_spec((DOUT, DOUT)),
        _full_spec((1, DOUT)),
    ],
    out_specs=_rows_spec(DOUT),
    out_shape=jax.ShapeDtypeStruct((N_PAD, DOUT), jnp.float32),
    compiler_params=_tc_params,
)

_tc_last = pl.pallas_call(
    _tc_last_body,
    grid=(TC_GRID,),
    in_specs=[
        _rows_spec(DOUT),
        _rows_spec(DOUT),
        _rows_spec(1),
        _full_spec((1, DOUT)),
    ],
    out_specs=_rows_spec(DOUT),
    out_shape=jax.ShapeDtypeStruct((N_PAD, DOUT), jnp.float32),
    compiler_params=_tc_params,
)


@jax.jit
def kernel(x, edge_index, W1, b1, W2, b2):
  pad_e = E_PAD - E
  src = jnp.concatenate(
      [edge_index[0], jnp.full((pad_e,), N, jnp.int32)]).reshape(E_ROWS, 128)
  dst = jnp.concatenate(
      [edge_index[1], jnp.full((pad_e,), N, jnp.int32)]).reshape(E_ROWS, 128)
  x_pad = jnp.pad(x, ((0, N_PAD - N), (0, 0)))

  dst2 = _tc_remap(dst)
  deg = _sc_degree(dst)
  d0 = deg[0].reshape(N_PAD, 1)
  d1 = deg[1].reshape(N_PAD, 1)

  hs1, dis = _tc_first(x_pad, W1, d0, d1)

  acc1 = _sc_agg(hs1, src, dst2).reshape(N_PAD, DOUT)
  hs2 = _tc_mid(acc1, hs1, dis, W2, b1.reshape(1, DOUT))

  acc2 = _sc_agg(hs2, src, dst2).reshape(N_PAD, DOUT)
  out = _tc_last(acc2, hs2, dis, b2.reshape(1, DOUT))
  return out[:N]


# speedup vs baseline: 16.0198x; 1.0019x over previous
import functools

import jax
import jax.numpy as jnp
from jax import lax
from jax.experimental import pallas as pl
from jax.experimental.pallas import tpu as pltpu
from jax.experimental.pallas import tpu_sc as plsc

N = 100000
E = 3200000
DIN = 128
DOUT = 16

NC = 2
NS = 16

ROW_BLK = 2048
N_PAD = 100352
TC_GRID = N_PAD // ROW_BLK
H = N_PAD // 2
LTRASH = H
H_ALL = H + 16
HSEG = H // NS
DSEG = N_PAD // NS
ZCH = 784

E_PAD = 2048 * 32 * 50
IDXN = 2048
EPT = E_PAD // NS
NCHUNK = EPT // IDXN
DEPT = E_PAD // (NC * NS)
DNCHUNK = DEPT // IDXN

_mesh = plsc.VectorSubcoreMesh(
    core_axis_name="c", subcore_axis_name="s", num_cores=NC, num_subcores=NS
)
_sc_params = pltpu.CompilerParams(use_tc_tiling_on_sc=False)


def _fill(ref, nvec, value):
  v = jnp.full((16,), value, jnp.float32)
  if len(ref.shape) == 1:
    def body(i, _):
      ref[pl.ds(i * 16, 16)] = v
      return 0
  else:
    def body(i, _):
      ref[i, :] = v
      return 0
  lax.fori_loop(0, nvec, body, 0)


@functools.partial(
    pl.kernel,
    out_type=jax.ShapeDtypeStruct((NC, N_PAD), jnp.float32),
    mesh=_mesh,
    scratch_types=[
        pltpu.VMEM((IDXN,), jnp.int32),
        pltpu.VMEM((IDXN,), jnp.float32),
        pltpu.VMEM((DSEG,), jnp.float32),
        pltpu.VMEM_SHARED((N_PAD,), jnp.float32),
    ],
    compiler_params=_sc_params,
)
def _sc_degree(dst_hbm, deg_hbm, idx_v, ones_v, stage_v, deg_sp):
  cid = lax.axis_index("c")
  sid = lax.axis_index("s")
  wid = cid * NS + sid

  _fill(ones_v, IDXN // 16, 1.0)
  _fill(stage_v, DSEG // 16, 0.0)
  pltpu.sync_copy(stage_v, deg_sp.at[pl.ds(sid * DSEG, DSEG)])
  plsc.subcore_barrier()

  tbase = wid * DEPT

  def chunk_body(ci, _):
    pltpu.sync_copy(dst_hbm.at[pl.ds(tbase + ci * IDXN, IDXN)], idx_v)
    pltpu.sync_copy(ones_v, deg_sp.at[idx_v], add=True)
    return 0

  lax.fori_loop(0, DNCHUNK, chunk_body, 0)
  plsc.subcore_barrier()

  pltpu.sync_copy(deg_sp.at[pl.ds(sid * DSEG, DSEG)], stage_v)
  pltpu.sync_copy(stage_v, deg_hbm.at[cid, pl.ds(sid * DSEG, DSEG)])


@functools.partial(
    pl.kernel,
    out_type=jax.ShapeDtypeStruct((NC, H, DOUT), jnp.float32),
    mesh=_mesh,
    scratch_types=[
        pltpu.VMEM((IDXN,), jnp.int32),
        pltpu.VMEM((IDXN,), jnp.int32),
        pltpu.VMEM((IDXN, DOUT), jnp.float32),
        pltpu.VMEM((ZCH, DOUT), jnp.float32),
        pltpu.VMEM_SHARED((H_ALL, DOUT), jnp.float32),
        pltpu.SemaphoreType.DMA,
    ],
    compiler_params=_sc_params,
)
def _sc_agg(hs_hbm, src_hbm, dst2_hbm, acc_hbm,
            src_v, dst_v, rows_v, stage_v, acc_sp, sem):
  cid = lax.axis_index("c")
  sid = lax.axis_index("s")

  _fill(stage_v, ZCH, 0.0)
  for k in range(4):
    pltpu.sync_copy(stage_v, acc_sp.at[pl.ds(sid * HSEG + k * ZCH, ZCH)])

  @pl.when(sid == 0)
  def _():
    pltpu.sync_copy(stage_v.at[pl.ds(0, 16)], acc_sp.at[pl.ds(H, 16)])

  plsc.subcore_barrier()

  tbase = sid * EPT

  def chunk_body(ci, _):
    r0 = tbase + ci * IDXN
    pltpu.sync_copy(src_hbm.at[pl.ds(r0, IDXN)], src_v)
    pltpu.sync_copy(dst2_hbm.at[cid, pl.ds(r0, IDXN)], dst_v)
    pltpu.async_copy(hs_hbm.at[src_v], rows_v, sem).wait()
    pltpu.sync_copy(rows_v, acc_sp.at[dst_v], add=True)
    return 0

  lax.fori_loop(0, NCHUNK, chunk_body, 0)
  plsc.subcore_barrier()

  for k in range(4):
    pltpu.sync_copy(acc_sp.at[pl.ds(sid * HSEG + k * ZCH, ZCH)], stage_v)
    pltpu.sync_copy(stage_v, acc_hbm.at[cid, pl.ds(sid * HSEG + k * ZCH, ZCH)])


def _tc_remap_body(dst_ref, out_ref):
  d = dst_ref[...]
  out_ref[0, :, :] = jnp.where(d < H, d, LTRASH)
  out_ref[1, :, :] = jnp.where(d >= H, d - H, LTRASH)


def _tc_first_body(x_ref, w_ref, d0_ref, d1_ref, hs_ref, dis_ref):
  z = jnp.dot(x_ref[...], w_ref[...], preferred_element_type=jnp.float32)
  d = d0_ref[...] + d1_ref[...] + 1.0
  dis = jnp.where(d > 0, lax.rsqrt(jnp.maximum(d, 1e-12)), 0.0)
  hs_ref[...] = z * dis
  dis_ref[...] = dis


def _tc_mid_body(a_ref, hs_ref, dis_ref, w_ref, b_ref, out_ref):
  dis = dis_ref[...]
  h = dis * (a_ref[...] + hs_ref[...]) + b_ref[...]
  h = jnp.maximum(h, 0.0)
  out_ref[...] = jnp.dot(h, w_ref[...], preferred_element_type=jnp.float32) * dis


def _tc_last_body(a_ref, hs_ref, dis_ref, b_ref, out_ref):
  out_ref[...] = dis_ref[...] * (a_ref[...] + hs_ref[...]) + b_ref[...]


def _rows_spec(width):
  return pl.BlockSpec((ROW_BLK, width), lambda i: (i, 0))


def _full_spec(shape):
  return pl.BlockSpec(shape, lambda i: tuple(0 for _ in shape))


_tc_params = pltpu.CompilerParams(dimension_semantics=("arbitrary",))

EBLK = 512
_tc_remap = pl.pallas_call(
    _tc_remap_body,
    grid=(E_PAD // 128 // EBLK,),
    in_specs=[pl.BlockSpec((EBLK, 128), lambda i: (i, 0))],
    out_specs=pl.BlockSpec((2, EBLK, 128), lambda i: (0, i, 0)),
    out_shape=jax.ShapeDtypeStruct((2, E_PAD // 128, 128), jnp.int32),
    compiler_params=_tc_params,
)

_tc_first = pl.pallas_call(
    _tc_first_body,
    grid=(TC_GRID,),
    in_specs=[
        _rows_spec(DIN),
        _full_spec((DIN, DOUT)),
        _rows_spec(1),
        _rows_spec(1),
    ],
    out_specs=[_rows_spec(DOUT), _rows_spec(1)],
    out_shape=[
        jax.ShapeDtypeStruct((N_PAD, DOUT), jnp.float32),
        jax.ShapeDtypeStruct((N_PAD, 1), jnp.float32),
    ],
    compiler_params=_tc_params,
)

_tc_mid = pl.pallas_call(
    _tc_mid_body,
    grid=(TC_GRID,),
    in_specs=[
        _rows_spec(DOUT),
        _rows_spec(DOUT),
        _rows_spec(1),
        _full_spec((DOUT, DOUT)),
        _full_spec((1, DOUT)),
    ],
    out_specs=_rows_spec(DOUT),
    out_shape=jax.ShapeDtypeStruct((N_PAD, DOUT), jnp.float32),
    compiler_params=_tc_params,
)

_tc_last = pl.pallas_call(
    _tc_last_body,
    grid=(TC_GRID,),
    in_specs=[
        _rows_spec(DOUT),
        _rows_spec(DOUT),
        _rows_spec(1),
        _full_spec((1, DOUT)),
    ],
    out_specs=_rows_spec(DOUT),
    out_shape=jax.ShapeDtypeStruct((N_PAD, DOUT), jnp.float32),
    compiler_params=_tc_params,
)


@jax.jit
def kernel(x, edge_index, W1, b1, W2, b2):
  pad_e = E_PAD - E
  src = jnp.concatenate(
      [edge_index[0], jnp.full((pad_e,), N, jnp.int32)])
  dst = jnp.concatenate(
      [edge_index[1], jnp.full((pad_e,), N, jnp.int32)])
  x_pad = jnp.pad(x, ((0, N_PAD - N), (0, 0)))

  dst2 = _tc_remap(dst.reshape(E_PAD // 128, 128)).reshape(2, E_PAD)
  deg = _sc_degree(dst)
  d0 = deg[0].reshape(N_PAD, 1)
  d1 = deg[1].reshape(N_PAD, 1)

  hs1, dis = _tc_first(x_pad, W1, d0, d1)

  acc1 = _sc_agg(hs1, src, dst2).reshape(N_PAD, DOUT)
  hs2 = _tc_mid(acc1, hs1, dis, W2, b1.reshape(1, DOUT))

  acc2 = _sc_agg(hs2, src, dst2).reshape(N_PAD, DOUT)
  out = _tc_last(acc2, hs2, dis, b2.reshape(1, DOUT))
  return out[:N]
